# baseline (speedup 1.0000x reference)
import functools

import jax
import jax.numpy as jnp
from jax import lax
from jax.experimental import pallas as pl
from jax.experimental.pallas import tpu as pltpu
from jax.experimental.pallas import tpu_sc as plsc

DIM = 64
BATCH = 16384
LANES = 16
NUM_CORES = 2
NUM_SUBCORES = 16
NUM_WORKERS = NUM_CORES * NUM_SUBCORES
ROWS_PER_W = BATCH // NUM_WORKERS
CHUNK = 128
NUM_CHUNKS = ROWS_PER_W // CHUNK
NUM_GROUPS = DIM // LANES


def _score_kernel(h_hbm, r_hbm, t_hbm, ere_hbm, eim_hbm, rre_hbm, rim_hbm,
                  out_hbm,
                  hidx_v, ridx_v, tidx_v,
                  hr_v, hi_v, tr_v, ti_v, rr_v, ri_v,
                  out_v, sem):
    wid = lax.axis_index("s") * NUM_CORES + lax.axis_index("c")
    base = wid * ROWS_PER_W

    pltpu.sync_copy(h_hbm.at[pl.ds(base, ROWS_PER_W)], hidx_v)
    pltpu.sync_copy(r_hbm.at[pl.ds(base, ROWS_PER_W)], ridx_v)
    pltpu.sync_copy(t_hbm.at[pl.ds(base, ROWS_PER_W)], tidx_v)

    for c in range(NUM_CHUNKS):
        off = c * CHUNK
        hsl = hidx_v.at[pl.ds(off, CHUNK)]
        rsl = ridx_v.at[pl.ds(off, CHUNK)]
        tsl = tidx_v.at[pl.ds(off, CHUNK)]
        copies = [
            pltpu.async_copy(ere_hbm.at[hsl], hr_v, sem),
            pltpu.async_copy(eim_hbm.at[hsl], hi_v, sem),
            pltpu.async_copy(ere_hbm.at[tsl], tr_v, sem),
            pltpu.async_copy(eim_hbm.at[tsl], ti_v, sem),
            pltpu.async_copy(rre_hbm.at[rsl], rr_v, sem),
            pltpu.async_copy(rim_hbm.at[rsl], ri_v, sem),
        ]
        for cp in copies:
            cp.wait()

        lane_iota = lax.iota(jnp.int32, LANES)

        def group_body(g16, _, off=off):
            rows = g16 * LANES + lane_iota

            def dim_body(d, acc):
                cols = jnp.full((LANES,), 0, jnp.int32) + d
                idx = [rows, cols]
                hr = plsc.load_gather(hr_v, idx)
                hi = plsc.load_gather(hi_v, idx)
                tr = plsc.load_gather(tr_v, idx)
                ti = plsc.load_gather(ti_v, idx)
                rr = plsc.load_gather(rr_v, idx)
                ri = plsc.load_gather(ri_v, idx)
                return (acc + rr * (hr * tr + hi * ti)
                        + ri * (hr * ti - hi * tr))

            scores = lax.fori_loop(0, DIM, dim_body,
                                   jnp.zeros((LANES,), jnp.float32))
            out_v[pl.ds(off + g16 * LANES, LANES)] = scores
            return 0

        lax.fori_loop(0, CHUNK // LANES, group_body, 0)

    pltpu.sync_copy(out_v, out_hbm.at[pl.ds(base, ROWS_PER_W)])


TBLK = 512


def _transpose_body(xre_ref, xim_ref, ore_ref, oim_ref):
    ore_ref[...] = xre_ref[...].T
    oim_ref[...] = xim_ref[...].T


def _to_row_major(ere_t, eim_t):
    n = ere_t.shape[1]
    grid = (n + TBLK - 1) // TBLK
    return pl.pallas_call(
        _transpose_body,
        grid=(grid,),
        in_specs=[
            pl.BlockSpec((DIM, TBLK), lambda j: (0, j)),
            pl.BlockSpec((DIM, TBLK), lambda j: (0, j)),
        ],
        out_specs=[
            pl.BlockSpec((TBLK, DIM), lambda j: (j, 0)),
            pl.BlockSpec((TBLK, DIM), lambda j: (j, 0)),
        ],
        out_shape=[
            jax.ShapeDtypeStruct((n, DIM), jnp.float32),
            jax.ShapeDtypeStruct((n, DIM), jnp.float32),
        ],
    )(ere_t, eim_t)


@functools.partial(jax.jit)
def _score(h, r, t, entity_re, entity_im, rel_re, rel_im):
    mesh = plsc.VectorSubcoreMesh(core_axis_name="c", subcore_axis_name="s")
    kern = functools.partial(
        pl.kernel,
        mesh=mesh,
        out_type=jax.ShapeDtypeStruct((BATCH,), jnp.float32),
        compiler_params=pltpu.CompilerParams(
            needs_layout_passes=False, use_tc_tiling_on_sc=False),
        scratch_types=[
            pltpu.VMEM((ROWS_PER_W,), jnp.int32),
            pltpu.VMEM((ROWS_PER_W,), jnp.int32),
            pltpu.VMEM((ROWS_PER_W,), jnp.int32),
            pltpu.VMEM((CHUNK, DIM), jnp.float32),
            pltpu.VMEM((CHUNK, DIM), jnp.float32),
            pltpu.VMEM((CHUNK, DIM), jnp.float32),
            pltpu.VMEM((CHUNK, DIM), jnp.float32),
            pltpu.VMEM((CHUNK, DIM), jnp.float32),
            pltpu.VMEM((CHUNK, DIM), jnp.float32),
            pltpu.VMEM((ROWS_PER_W,), jnp.float32),
            pltpu.SemaphoreType.DMA,
        ],
    )(_score_kernel)
    ere_rm, eim_rm = _to_row_major(entity_re.T, entity_im.T)
    return kern(h, r, t, ere_rm, eim_rm, rel_re, rel_im)


def kernel(h, r, t, entity_re, entity_im, rel_re, rel_im):
    return _score(h, r, t, entity_re, entity_im, rel_re, rel_im)


# speedup vs baseline: 1.1942x; 1.1942x over previous
import functools

import jax
import jax.numpy as jnp
from jax import lax
from jax.experimental import pallas as pl
from jax.experimental.pallas import tpu as pltpu
from jax.experimental.pallas import tpu_sc as plsc

DIM = 64
BATCH = 16384
LANES = 16
NUM_CORES = 2
NUM_SUBCORES = 16
NUM_WORKERS = NUM_CORES * NUM_SUBCORES
ROWS_PER_W = BATCH // NUM_WORKERS
CHUNK = 128
NUM_CHUNKS = ROWS_PER_W // CHUNK
NUM_GROUPS = DIM // LANES


def _score_kernel(h_hbm, r_hbm, t_hbm, ere_hbm, eim_hbm, rre_hbm, rim_hbm,
                  out_hbm,
                  hidx_v, ridx_v, tidx_v,
                  hr_v, hi_v, tr_v, ti_v, rr_v, ri_v,
                  out_v, sem):
    wid = lax.axis_index("s") * NUM_CORES + lax.axis_index("c")
    base = wid * ROWS_PER_W

    pltpu.sync_copy(h_hbm.at[pl.ds(base, ROWS_PER_W)], hidx_v)
    pltpu.sync_copy(r_hbm.at[pl.ds(base, ROWS_PER_W)], ridx_v)
    pltpu.sync_copy(t_hbm.at[pl.ds(base, ROWS_PER_W)], tidx_v)

    for c in range(NUM_CHUNKS):
        off = c * CHUNK
        hsl = hidx_v.at[pl.ds(off, CHUNK)]
        rsl = ridx_v.at[pl.ds(off, CHUNK)]
        tsl = tidx_v.at[pl.ds(off, CHUNK)]
        copies = [
            pltpu.async_copy(ere_hbm.at[hsl], hr_v, sem),
            pltpu.async_copy(eim_hbm.at[hsl], hi_v, sem),
            pltpu.async_copy(ere_hbm.at[tsl], tr_v, sem),
            pltpu.async_copy(eim_hbm.at[tsl], ti_v, sem),
            pltpu.async_copy(rre_hbm.at[rsl], rr_v, sem),
            pltpu.async_copy(rim_hbm.at[rsl], ri_v, sem),
        ]
        for cp in copies:
            cp.wait()

        lane_iota = lax.iota(jnp.int32, LANES)

        def group_body(g16, _, off=off):
            rows = g16 * LANES + lane_iota

            def dim_body(d, acc):
                cols = jnp.full((LANES,), 0, jnp.int32) + d
                idx = [rows, cols]
                hr = plsc.load_gather(hr_v, idx)
                hi = plsc.load_gather(hi_v, idx)
                tr = plsc.load_gather(tr_v, idx)
                ti = plsc.load_gather(ti_v, idx)
                rr = plsc.load_gather(rr_v, idx)
                ri = plsc.load_gather(ri_v, idx)
                return (acc + rr * (hr * tr + hi * ti)
                        + ri * (hr * ti - hi * tr))

            scores = lax.fori_loop(0, DIM, dim_body,
                                   jnp.zeros((LANES,), jnp.float32))
            out_v[pl.ds(off + g16 * LANES, LANES)] = scores
            return 0

        lax.fori_loop(0, CHUNK // LANES, group_body, 0)

    pltpu.sync_copy(out_v, out_hbm.at[pl.ds(base, ROWS_PER_W)])


TBLK = 2048


def _transpose_body(xre_ref, xim_ref, ore_ref, oim_ref):
    eye = jnp.eye(DIM, dtype=jnp.float32)
    dims = (((0,), (0,)), ((), ()))
    ore_ref[...] = lax.dot_general(xre_ref[...], eye, dims,
                                   precision=lax.Precision.HIGHEST)
    oim_ref[...] = lax.dot_general(xim_ref[...], eye, dims,
                                   precision=lax.Precision.HIGHEST)


def _to_row_major(ere_t, eim_t):
    n = ere_t.shape[1]
    grid = (n + TBLK - 1) // TBLK
    return pl.pallas_call(
        _transpose_body,
        grid=(grid,),
        in_specs=[
            pl.BlockSpec((DIM, TBLK), lambda j: (0, j)),
            pl.BlockSpec((DIM, TBLK), lambda j: (0, j)),
        ],
        out_specs=[
            pl.BlockSpec((TBLK, DIM), lambda j: (j, 0)),
            pl.BlockSpec((TBLK, DIM), lambda j: (j, 0)),
        ],
        out_shape=[
            jax.ShapeDtypeStruct((n, DIM), jnp.float32),
            jax.ShapeDtypeStruct((n, DIM), jnp.float32),
        ],
    )(ere_t, eim_t)


@functools.partial(jax.jit)
def _score(h, r, t, entity_re, entity_im, rel_re, rel_im):
    mesh = plsc.VectorSubcoreMesh(core_axis_name="c", subcore_axis_name="s")
    kern = functools.partial(
        pl.kernel,
        mesh=mesh,
        out_type=jax.ShapeDtypeStruct((BATCH,), jnp.float32),
        compiler_params=pltpu.CompilerParams(
            needs_layout_passes=False, use_tc_tiling_on_sc=False),
        scratch_types=[
            pltpu.VMEM((ROWS_PER_W,), jnp.int32),
            pltpu.VMEM((ROWS_PER_W,), jnp.int32),
            pltpu.VMEM((ROWS_PER_W,), jnp.int32),
            pltpu.VMEM((CHUNK, DIM), jnp.float32),
            pltpu.VMEM((CHUNK, DIM), jnp.float32),
            pltpu.VMEM((CHUNK, DIM), jnp.float32),
            pltpu.VMEM((CHUNK, DIM), jnp.float32),
            pltpu.VMEM((CHUNK, DIM), jnp.float32),
            pltpu.VMEM((CHUNK, DIM), jnp.float32),
            pltpu.VMEM((ROWS_PER_W,), jnp.float32),
            pltpu.SemaphoreType.DMA,
        ],
    )(_score_kernel)
    ere_rm, eim_rm = _to_row_major(entity_re.T, entity_im.T)
    return kern(h, r, t, ere_rm, eim_rm, rel_re, rel_im)


def kernel(h, r, t, entity_re, entity_im, rel_re, rel_im):
    return _score(h, r, t, entity_re, entity_im, rel_re, rel_im)


# speedup vs baseline: 1.4754x; 1.2356x over previous
import functools

import jax
import jax.numpy as jnp
from jax import lax
from jax.experimental import pallas as pl
from jax.experimental.pallas import tpu as pltpu
from jax.experimental.pallas import tpu_sc as plsc

DIM = 64
BATCH = 16384
LANES = 16
NUM_CORES = 2
NUM_SUBCORES = 16
NUM_WORKERS = NUM_CORES * NUM_SUBCORES
ROWS_PER_W = BATCH // NUM_WORKERS
CHUNK = 128
NUM_CHUNKS = ROWS_PER_W // CHUNK
NUM_GROUPS = DIM // LANES


def _score_kernel(h_hbm, r_hbm, t_hbm, ere_hbm, eim_hbm, rre_hbm, rim_hbm,
                  out_hbm,
                  hidx_v, ridx_v, tidx_v,
                  hr_v, hi_v, tr_v, ti_v, rr_v, ri_v,
                  out_v, sem):
    wid = lax.axis_index("s") * NUM_CORES + lax.axis_index("c")
    base = wid * ROWS_PER_W

    pltpu.sync_copy(h_hbm.at[pl.ds(base, ROWS_PER_W)], hidx_v)
    pltpu.sync_copy(r_hbm.at[pl.ds(base, ROWS_PER_W)], ridx_v)
    pltpu.sync_copy(t_hbm.at[pl.ds(base, ROWS_PER_W)], tidx_v)

    for c in range(NUM_CHUNKS):
        off = c * CHUNK
        hsl = hidx_v.at[pl.ds(off, CHUNK)]
        rsl = ridx_v.at[pl.ds(off, CHUNK)]
        tsl = tidx_v.at[pl.ds(off, CHUNK)]
        copies = [
            pltpu.async_copy(ere_hbm.at[hsl], hr_v, sem),
            pltpu.async_copy(eim_hbm.at[hsl], hi_v, sem),
            pltpu.async_copy(ere_hbm.at[tsl], tr_v, sem),
            pltpu.async_copy(eim_hbm.at[tsl], ti_v, sem),
            pltpu.async_copy(rre_hbm.at[rsl], rr_v, sem),
            pltpu.async_copy(rim_hbm.at[rsl], ri_v, sem),
        ]
        for cp in copies:
            cp.wait()

        lane_iota = lax.iota(jnp.int32, LANES)

        def group_body(g16, _, off=off):
            rows = g16 * LANES + lane_iota

            def dim_body(d, acc):
                cols = jnp.full((LANES,), 0, jnp.int32) + d
                idx = [rows, cols]
                hr = plsc.load_gather(hr_v, idx)
                hi = plsc.load_gather(hi_v, idx)
                tr = plsc.load_gather(tr_v, idx)
                ti = plsc.load_gather(ti_v, idx)
                rr = plsc.load_gather(rr_v, idx)
                ri = plsc.load_gather(ri_v, idx)
                return (acc + rr * (hr * tr + hi * ti)
                        + ri * (hr * ti - hi * tr))

            scores = lax.fori_loop(0, DIM, dim_body,
                                   jnp.zeros((LANES,), jnp.float32))
            out_v[pl.ds(off + g16 * LANES, LANES)] = scores
            return 0

        lax.fori_loop(0, CHUNK // LANES, group_body, 0)

    pltpu.sync_copy(out_v, out_hbm.at[pl.ds(base, ROWS_PER_W)])


TBLK = 2048


def _transpose_body(xre_ref, xim_ref, ore_ref, oim_ref):
    ore_ref[...] = xre_ref[...].T
    oim_ref[...] = xim_ref[...].T


def _to_row_major(ere_t, eim_t):
    n = ere_t.shape[1]
    grid = (n + TBLK - 1) // TBLK
    return pl.pallas_call(
        _transpose_body,
        grid=(grid,),
        in_specs=[
            pl.BlockSpec((DIM, TBLK), lambda j: (0, j)),
            pl.BlockSpec((DIM, TBLK), lambda j: (0, j)),
        ],
        out_specs=[
            pl.BlockSpec((TBLK, DIM), lambda j: (j, 0)),
            pl.BlockSpec((TBLK, DIM), lambda j: (j, 0)),
        ],
        out_shape=[
            jax.ShapeDtypeStruct((n, DIM), jnp.float32),
            jax.ShapeDtypeStruct((n, DIM), jnp.float32),
        ],
    )(ere_t, eim_t)


@functools.partial(jax.jit)
def _score(h, r, t, entity_re, entity_im, rel_re, rel_im):
    mesh = plsc.VectorSubcoreMesh(core_axis_name="c", subcore_axis_name="s")
    kern = functools.partial(
        pl.kernel,
        mesh=mesh,
        out_type=jax.ShapeDtypeStruct((BATCH,), jnp.float32),
        compiler_params=pltpu.CompilerParams(
            needs_layout_passes=False, use_tc_tiling_on_sc=False),
        scratch_types=[
            pltpu.VMEM((ROWS_PER_W,), jnp.int32),
            pltpu.VMEM((ROWS_PER_W,), jnp.int32),
            pltpu.VMEM((ROWS_PER_W,), jnp.int32),
            pltpu.VMEM((CHUNK, DIM), jnp.float32),
            pltpu.VMEM((CHUNK, DIM), jnp.float32),
            pltpu.VMEM((CHUNK, DIM), jnp.float32),
            pltpu.VMEM((CHUNK, DIM), jnp.float32),
            pltpu.VMEM((CHUNK, DIM), jnp.float32),
            pltpu.VMEM((CHUNK, DIM), jnp.float32),
            pltpu.VMEM((ROWS_PER_W,), jnp.float32),
            pltpu.SemaphoreType.DMA,
        ],
    )(_score_kernel)
    ere_rm, eim_rm = _to_row_major(entity_re.T, entity_im.T)
    return kern(h, r, t, ere_rm, eim_rm, rel_re, rel_im)


def kernel(h, r, t, entity_re, entity_im, rel_re, rel_im):
    return _score(h, r, t, entity_re, entity_im, rel_re, rel_im)


# speedup vs baseline: 1.6750x; 1.1352x over previous
import functools

import jax
import jax.numpy as jnp
from jax import lax
from jax.experimental import pallas as pl
from jax.experimental.pallas import tpu as pltpu
from jax.experimental.pallas import tpu_sc as plsc

DIM = 64
BATCH = 16384
LANES = 16
NUM_CORES = 2
NUM_SUBCORES = 16
NUM_WORKERS = NUM_CORES * NUM_SUBCORES
ROWS_PER_W = BATCH // NUM_WORKERS
CHUNK = 128
NUM_CHUNKS = ROWS_PER_W // CHUNK
NUM_GROUPS = DIM // LANES


def _score_kernel(h_hbm, r_hbm, t_hbm, ere_hbm, eim_hbm, rre_hbm, rim_hbm,
                  out_hbm,
                  hidx_v, ridx_v, tidx_v,
                  hr_v, hi_v, tr_v, ti_v, rr_v, ri_v,
                  out_v, sem):
    wid = lax.axis_index("s") * NUM_CORES + lax.axis_index("c")
    base = wid * ROWS_PER_W

    pltpu.sync_copy(h_hbm.at[pl.ds(base, ROWS_PER_W)], hidx_v)
    pltpu.sync_copy(r_hbm.at[pl.ds(base, ROWS_PER_W)], ridx_v)
    pltpu.sync_copy(t_hbm.at[pl.ds(base, ROWS_PER_W)], tidx_v)

    for c in range(NUM_CHUNKS):
        off = c * CHUNK
        hsl = hidx_v.at[pl.ds(off, CHUNK)]
        rsl = ridx_v.at[pl.ds(off, CHUNK)]
        tsl = tidx_v.at[pl.ds(off, CHUNK)]
        copies = [
            pltpu.async_copy(ere_hbm.at[hsl], hr_v, sem),
            pltpu.async_copy(eim_hbm.at[hsl], hi_v, sem),
            pltpu.async_copy(ere_hbm.at[tsl], tr_v, sem),
            pltpu.async_copy(eim_hbm.at[tsl], ti_v, sem),
            pltpu.async_copy(rre_hbm.at[rsl], rr_v, sem),
            pltpu.async_copy(rim_hbm.at[rsl], ri_v, sem),
        ]
        for cp in copies:
            cp.wait()

        lane_iota = lax.iota(jnp.int32, LANES)

        def group_body(g16, _, off=off):
            rows = g16 * LANES + lane_iota

            def dim_body(d, acc):
                cols = jnp.full((LANES,), 0, jnp.int32) + d
                idx = [rows, cols]
                hr = plsc.load_gather(hr_v, idx)
                hi = plsc.load_gather(hi_v, idx)
                tr = plsc.load_gather(tr_v, idx)
                ti = plsc.load_gather(ti_v, idx)
                rr = plsc.load_gather(rr_v, idx)
                ri = plsc.load_gather(ri_v, idx)
                return (acc + rr * (hr * tr + hi * ti)
                        + ri * (hr * ti - hi * tr))

            scores = lax.fori_loop(0, DIM, dim_body,
                                   jnp.zeros((LANES,), jnp.float32))
            out_v[pl.ds(off + g16 * LANES, LANES)] = scores
            return 0

        lax.fori_loop(0, CHUNK // LANES, group_body, 0)

    pltpu.sync_copy(out_v, out_hbm.at[pl.ds(base, ROWS_PER_W)])


TBLK = 8192


def _transpose_body(xre_ref, xim_ref, ore_ref, oim_ref):
    ore_ref[...] = xre_ref[...].T
    oim_ref[...] = xim_ref[...].T


def _to_row_major(ere_t, eim_t):
    n = ere_t.shape[1]
    grid = (n + TBLK - 1) // TBLK
    return pl.pallas_call(
        _transpose_body,
        grid=(grid,),
        in_specs=[
            pl.BlockSpec((DIM, TBLK), lambda j: (0, j)),
            pl.BlockSpec((DIM, TBLK), lambda j: (0, j)),
        ],
        out_specs=[
            pl.BlockSpec((TBLK, DIM), lambda j: (j, 0)),
            pl.BlockSpec((TBLK, DIM), lambda j: (j, 0)),
        ],
        out_shape=[
            jax.ShapeDtypeStruct((n, DIM), jnp.float32),
            jax.ShapeDtypeStruct((n, DIM), jnp.float32),
        ],
    )(ere_t, eim_t)


@functools.partial(jax.jit)
def _score(h, r, t, entity_re, entity_im, rel_re, rel_im):
    mesh = plsc.VectorSubcoreMesh(core_axis_name="c", subcore_axis_name="s")
    kern = functools.partial(
        pl.kernel,
        mesh=mesh,
        out_type=jax.ShapeDtypeStruct((BATCH,), jnp.float32),
        compiler_params=pltpu.CompilerParams(
            needs_layout_passes=False, use_tc_tiling_on_sc=False),
        scratch_types=[
            pltpu.VMEM((ROWS_PER_W,), jnp.int32),
            pltpu.VMEM((ROWS_PER_W,), jnp.int32),
            pltpu.VMEM((ROWS_PER_W,), jnp.int32),
            pltpu.VMEM((CHUNK, DIM), jnp.float32),
            pltpu.VMEM((CHUNK, DIM), jnp.float32),
            pltpu.VMEM((CHUNK, DIM), jnp.float32),
            pltpu.VMEM((CHUNK, DIM), jnp.float32),
            pltpu.VMEM((CHUNK, DIM), jnp.float32),
            pltpu.VMEM((CHUNK, DIM), jnp.float32),
            pltpu.VMEM((ROWS_PER_W,), jnp.float32),
            pltpu.SemaphoreType.DMA,
        ],
    )(_score_kernel)
    ere_rm, eim_rm = _to_row_major(entity_re.T, entity_im.T)
    return kern(h, r, t, ere_rm, eim_rm, rel_re, rel_im)


def kernel(h, r, t, entity_re, entity_im, rel_re, rel_im):
    return _score(h, r, t, entity_re, entity_im, rel_re, rel_im)


# speedup vs baseline: 1.9222x; 1.1476x over previous
import functools

import jax
import jax.numpy as jnp
from jax import lax
from jax.experimental import pallas as pl
from jax.experimental.pallas import tpu as pltpu
from jax.experimental.pallas import tpu_sc as plsc

DIM = 64
BATCH = 16384
LANES = 16
NUM_CORES = 2
NUM_SUBCORES = 16
NUM_WORKERS = NUM_CORES * NUM_SUBCORES
ROWS_PER_W = BATCH // NUM_WORKERS
CHUNK = 128
NUM_CHUNKS = ROWS_PER_W // CHUNK
NUM_GROUPS = DIM // LANES
NBUF = 2


def _fire(c, buf, h_hbm_s, r_hbm_s, t_hbm_s,
          ere_hbm, eim_hbm, rre_hbm, rim_hbm,
          hidx_v, ridx_v, tidx_v, ent_v, rel_v, sems):
    off = c * CHUNK
    hsl = hidx_v.at[pl.ds(off, CHUNK)]
    rsl = ridx_v.at[pl.ds(off, CHUNK)]
    tsl = tidx_v.at[pl.ds(off, CHUNK)]
    pltpu.async_copy(ere_hbm.at[hsl], ent_v.at[buf, 0], sems.at[buf])
    pltpu.async_copy(eim_hbm.at[hsl], ent_v.at[buf, 1], sems.at[buf])
    pltpu.async_copy(ere_hbm.at[tsl], ent_v.at[buf, 2], sems.at[buf])
    pltpu.async_copy(eim_hbm.at[tsl], ent_v.at[buf, 3], sems.at[buf])
    pltpu.async_copy(rre_hbm.at[rsl], rel_v.at[buf, 0], sems.at[buf])
    pltpu.async_copy(rim_hbm.at[rsl], rel_v.at[buf, 1], sems.at[buf])


def _drain(buf, ere_hbm, ent_v, rel_v, sems):
    for k in range(4):
        pltpu.make_async_copy(ere_hbm.at[pl.ds(0, CHUNK)], ent_v.at[buf, k],
                              sems.at[buf]).wait()
    for k in range(2):
        pltpu.make_async_copy(ere_hbm.at[pl.ds(0, CHUNK)], rel_v.at[buf, k],
                              sems.at[buf]).wait()


def _score_kernel(h_hbm, r_hbm, t_hbm, ere_hbm, eim_hbm, rre_hbm, rim_hbm,
                  out_hbm,
                  hidx_v, ridx_v, tidx_v,
                  ent_v, rel_v,
                  out_v, sems):
    wid = lax.axis_index("s") * NUM_CORES + lax.axis_index("c")
    base = wid * ROWS_PER_W

    pltpu.sync_copy(h_hbm.at[pl.ds(base, ROWS_PER_W)], hidx_v)
    pltpu.sync_copy(r_hbm.at[pl.ds(base, ROWS_PER_W)], ridx_v)
    pltpu.sync_copy(t_hbm.at[pl.ds(base, ROWS_PER_W)], tidx_v)

    args = (h_hbm, r_hbm, t_hbm, ere_hbm, eim_hbm, rre_hbm, rim_hbm,
            hidx_v, ridx_v, tidx_v, ent_v, rel_v, sems)
    _fire(0, 0, *args[:3], *args[3:7], *args[7:])

    lane_iota = lax.iota(jnp.int32, LANES)

    for c in range(NUM_CHUNKS):
        buf = c % NBUF
        _drain(buf, ere_hbm, ent_v, rel_v, sems)
        if c + 1 < NUM_CHUNKS:
            _fire(c + 1, (c + 1) % NBUF, *args[:3], *args[3:7], *args[7:])

        hr_v = ent_v.at[buf, 0]
        hi_v = ent_v.at[buf, 1]
        tr_v = ent_v.at[buf, 2]
        ti_v = ent_v.at[buf, 3]
        rr_v = rel_v.at[buf, 0]
        ri_v = rel_v.at[buf, 1]
        off = c * CHUNK

        def group_body(g16, _, hr_v=hr_v, hi_v=hi_v, tr_v=tr_v, ti_v=ti_v,
                       rr_v=rr_v, ri_v=ri_v, off=off):
            rows = g16 * LANES + lane_iota

            def dim_body(d, acc):
                cols = jnp.full((LANES,), 0, jnp.int32) + d
                idx = [rows, cols]
                hr = plsc.load_gather(hr_v, idx)
                hi = plsc.load_gather(hi_v, idx)
                tr = plsc.load_gather(tr_v, idx)
                ti = plsc.load_gather(ti_v, idx)
                rr = plsc.load_gather(rr_v, idx)
                ri = plsc.load_gather(ri_v, idx)
                return (acc + rr * (hr * tr + hi * ti)
                        + ri * (hr * ti - hi * tr))

            scores = lax.fori_loop(0, DIM, dim_body,
                                   jnp.zeros((LANES,), jnp.float32))
            out_v[pl.ds(off + g16 * LANES, LANES)] = scores
            return 0

        lax.fori_loop(0, CHUNK // LANES, group_body, 0)

    pltpu.sync_copy(out_v, out_hbm.at[pl.ds(base, ROWS_PER_W)])


@functools.partial(jax.jit)
def _score(h, r, t, entity_re, entity_im, rel_re, rel_im):
    mesh = plsc.VectorSubcoreMesh(core_axis_name="c", subcore_axis_name="s")
    kern = functools.partial(
        pl.kernel,
        mesh=mesh,
        out_type=jax.ShapeDtypeStruct((BATCH,), jnp.float32),
        compiler_params=pltpu.CompilerParams(
            needs_layout_passes=False, use_tc_tiling_on_sc=False),
        scratch_types=[
            pltpu.VMEM((ROWS_PER_W,), jnp.int32),
            pltpu.VMEM((ROWS_PER_W,), jnp.int32),
            pltpu.VMEM((ROWS_PER_W,), jnp.int32),
            pltpu.VMEM((NBUF, 4, CHUNK, DIM), jnp.float32),
            pltpu.VMEM((NBUF, 2, CHUNK, DIM), jnp.float32),
            pltpu.VMEM((ROWS_PER_W,), jnp.float32),
            pltpu.SemaphoreType.DMA((NBUF,)),
        ],
    )(_score_kernel)
    return kern(h, r, t, entity_re, entity_im, rel_re, rel_im)


def kernel(h, r, t, entity_re, entity_im, rel_re, rel_im):
    return _score(h, r, t, entity_re, entity_im, rel_re, rel_im)


# speedup vs baseline: 2.7441x; 1.4276x over previous
import functools

import jax
import jax.numpy as jnp
from jax import lax
from jax.experimental import pallas as pl
from jax.experimental.pallas import tpu as pltpu
from jax.experimental.pallas import tpu_sc as plsc

DIM = 64
BATCH = 16384
NENT = 1000000
LANES = 16
NUM_CORES = 2
NUM_SUBCORES = 16
NUM_WORKERS = NUM_CORES * NUM_SUBCORES
ROWS_PER_W = BATCH // NUM_WORKERS
CHUNK = 128
NUM_CHUNKS = ROWS_PER_W // CHUNK
NUM_GROUPS = DIM // LANES
NBUF = 2

BW = 256
NBLKS = (NENT + BW - 1) // BW
LAST_BW = NENT - (NBLKS - 1) * BW
BPT = (NBLKS + NUM_WORKERS - 1) // NUM_WORKERS
STCAP = 64
DUMMY = BATCH


def _p1_kernel(h_hbm, t_hbm, ere3, eim3,
               hst_hbm, tst_hbm,
               h_v, t_v, hlist_v, tlist_v,
               eb_v, ib_v, tmp_v,
               hstg_v, tstg_v, hbl_v, tbl_v):
    wid = lax.axis_index("s") * NUM_CORES + lax.axis_index("c")
    b0 = wid * BPT
    nblk = jnp.minimum(BPT, NBLKS - b0)
    lane_iota = lax.iota(jnp.int32, LANES)

    pltpu.sync_copy(h_hbm, h_v)
    pltpu.sync_copy(t_hbm, t_v)

    def build(arr_v, list_v):
        def sb(i, cnt):
            e = arr_v[pl.ds(i * LANES, LANES)]
            blk = e >> 8
            m = (blk >= b0) & (blk < b0 + nblk)
            cnt_i = jnp.sum(m.astype(jnp.int32))
            enc = ((e - b0 * BW) << 14) | (i * LANES + lane_iota)
            plsc.store_compressed(list_v.at[pl.ds(cnt, LANES)], enc, mask=m)
            return cnt + cnt_i

        return lax.fori_loop(0, BATCH // LANES, sb, 0)

    nh = build(h_v, hlist_v)
    nt = build(t_v, tlist_v)

    dummy_vec = jnp.full((LANES,), DUMMY, jnp.int32)
    for q in range(STCAP // LANES):
        hbl_v[pl.ds(q * LANES, LANES)] = dummy_vec
        tbl_v[pl.ds(q * LANES, LANES)] = dummy_vec

    def serve(list_v, n, j, stg_v, bl_v, out_hbm, cnt0):
        niter = (n + LANES - 1) >> 4

        def vs(i, cnt):
            encs = list_v[pl.ds(i * LANES, LANES)]
            valid = (i * LANES + lane_iota) < n
            mm = valid & ((encs >> 22) == j)
            cm = jnp.sum(mm.astype(jnp.int32))
            plsc.store_compressed(tmp_v.at[pl.ds(0, LANES)], encs, mask=mm)

            def hb(k, cnt):
                es = tmp_v[pl.ds(k, LANES)]
                enc_s = jnp.sum(jnp.where(lane_iota == 0, es, 0))
                b = enc_s & (BATCH - 1)
                col = (enc_s >> 14) & (BW - 1)
                colv = jnp.full((LANES,), 0, jnp.int32) + col
                slot = cnt & (STCAP - 1)
                for g in range(NUM_GROUPS):
                    d = g * LANES + lane_iota
                    idx = [d >> 3, d & 7, colv]
                    stg_v[slot, pl.ds(g * LANES, LANES)] = \
                        plsc.load_gather(eb_v, idx)
                    stg_v[slot, pl.ds(DIM + g * LANES, LANES)] = \
                        plsc.load_gather(ib_v, idx)
                q = slot >> 4
                rmod = slot & (LANES - 1)
                w = bl_v[pl.ds(q * LANES, LANES)]
                bl_v[pl.ds(q * LANES, LANES)] = \
                    jnp.where(lane_iota == rmod, b, w)
                cnt = cnt + 1

                @pl.when((cnt & (STCAP - 1)) == 0)
                def _flush():
                    pltpu.sync_copy(stg_v, out_hbm.at[bl_v])
                    for q2 in range(STCAP // LANES):
                        bl_v[pl.ds(q2 * LANES, LANES)] = dummy_vec

                return cnt

            return lax.fori_loop(0, cm, hb, cnt)

        return lax.fori_loop(0, niter, vs, cnt0)

    def bb(j, carry):
        ch, ct = carry
        jg = b0 + j

        @pl.when(jg < NBLKS - 1)
        def _full():
            pltpu.sync_copy(ere3.at[:, :, pl.ds(jg * BW, BW)], eb_v)
            pltpu.sync_copy(eim3.at[:, :, pl.ds(jg * BW, BW)], ib_v)

        @pl.when(jg == NBLKS - 1)
        def _last():
            pltpu.sync_copy(ere3.at[:, :, pl.ds(jg * BW, LAST_BW)],
                            eb_v.at[:, :, pl.ds(0, LAST_BW)])
            pltpu.sync_copy(eim3.at[:, :, pl.ds(jg * BW, LAST_BW)],
                            ib_v.at[:, :, pl.ds(0, LAST_BW)])

        ch = serve(hlist_v, nh, j, hstg_v, hbl_v, hst_hbm, ch)
        ct = serve(tlist_v, nt, j, tstg_v, tbl_v, tst_hbm, ct)
        return (ch, ct)

    ch, ct = lax.fori_loop(0, nblk, bb, (0, 0))

    @pl.when((ch & (STCAP - 1)) != 0)
    def _tail_h():
        pltpu.sync_copy(hstg_v, hst_hbm.at[hbl_v])

    @pl.when((ct & (STCAP - 1)) != 0)
    def _tail_t():
        pltpu.sync_copy(tstg_v, tst_hbm.at[tbl_v])


def _p2_fire(c, buf, base, hst_hbm, tst_hbm, rre_hbm, rim_hbm,
             ridx_v, ent_v, rel_v, sems):
    off = c * CHUNK
    rsl = ridx_v.at[pl.ds(off, CHUNK)]
    sl = pl.ds(base + off, CHUNK)
    pltpu.async_copy(hst_hbm.at[sl], ent_v.at[buf, 0], sems.at[buf])
    pltpu.async_copy(tst_hbm.at[sl], ent_v.at[buf, 1], sems.at[buf])
    pltpu.async_copy(rre_hbm.at[rsl], rel_v.at[buf, 0], sems.at[buf])
    pltpu.async_copy(rim_hbm.at[rsl], rel_v.at[buf, 1], sems.at[buf])


def _p2_drain(buf, hst_hbm, rre_hbm, ent_v, rel_v, sems):
    for k in range(2):
        pltpu.make_async_copy(hst_hbm.at[pl.ds(0, CHUNK)], ent_v.at[buf, k],
                              sems.at[buf]).wait()
    for k in range(2):
        pltpu.make_async_copy(rre_hbm.at[pl.ds(0, CHUNK)], rel_v.at[buf, k],
                              sems.at[buf]).wait()


def _p2_kernel(r_hbm, hst_hbm, tst_hbm, rre_hbm, rim_hbm,
               out_hbm, ridx_v, ent_v, rel_v, out_v, sems):
    wid = lax.axis_index("s") * NUM_CORES + lax.axis_index("c")
    base = wid * ROWS_PER_W
    lane_iota = lax.iota(jnp.int32, LANES)

    pltpu.sync_copy(r_hbm.at[pl.ds(base, ROWS_PER_W)], ridx_v)

    args = (base, hst_hbm, tst_hbm, rre_hbm, rim_hbm,
            ridx_v, ent_v, rel_v, sems)
    _p2_fire(0, 0, *args)

    for c in range(NUM_CHUNKS):
        buf = c % NBUF
        _p2_drain(buf, hst_hbm, rre_hbm, ent_v, rel_v, sems)
        if c + 1 < NUM_CHUNKS:
            _p2_fire(c + 1, (c + 1) % NBUF, *args)

        hv = ent_v.at[buf, 0]
        tv = ent_v.at[buf, 1]
        rr_v = rel_v.at[buf, 0]
        ri_v = rel_v.at[buf, 1]
        off = c * CHUNK

        def group_body(g16, _, hv=hv, tv=tv, rr_v=rr_v, ri_v=ri_v, off=off):
            rows = g16 * LANES + lane_iota

            def dim_body(d, acc):
                cols = jnp.full((LANES,), 0, jnp.int32) + d
                hr = plsc.load_gather(hv, [rows, cols])
                hi = plsc.load_gather(hv, [rows, cols + DIM])
                tr = plsc.load_gather(tv, [rows, cols])
                ti = plsc.load_gather(tv, [rows, cols + DIM])
                rr = plsc.load_gather(rr_v, [rows, cols])
                ri = plsc.load_gather(ri_v, [rows, cols])
                return (acc + rr * (hr * tr + hi * ti)
                        + ri * (hr * ti - hi * tr))

            scores = lax.fori_loop(0, DIM, dim_body,
                                   jnp.zeros((LANES,), jnp.float32))
            out_v[pl.ds(off + g16 * LANES, LANES)] = scores
            return 0

        lax.fori_loop(0, CHUNK // LANES, group_body, 0)

    pltpu.sync_copy(out_v, out_hbm.at[pl.ds(base, ROWS_PER_W)])


@functools.partial(jax.jit)
def _score(h, r, t, entity_re, entity_im, rel_re, rel_im):
    mesh = plsc.VectorSubcoreMesh(core_axis_name="c", subcore_axis_name="s")
    stage = jax.ShapeDtypeStruct((BATCH + STCAP, 2 * DIM), jnp.float32)
    p1 = functools.partial(
        pl.kernel,
        mesh=mesh,
        out_type=[stage, stage],
        compiler_params=pltpu.CompilerParams(needs_layout_passes=False),
        scratch_types=[
            pltpu.VMEM((BATCH,), jnp.int32),
            pltpu.VMEM((BATCH,), jnp.int32),
            pltpu.VMEM((BATCH + LANES,), jnp.int32),
            pltpu.VMEM((BATCH + LANES,), jnp.int32),
            pltpu.VMEM((8, 8, BW), jnp.float32),
            pltpu.VMEM((8, 8, BW), jnp.float32),
            pltpu.VMEM((2 * LANES - 1,), jnp.int32),
            pltpu.VMEM((STCAP, 2 * DIM), jnp.float32),
            pltpu.VMEM((STCAP, 2 * DIM), jnp.float32),
            pltpu.VMEM((STCAP,), jnp.int32),
            pltpu.VMEM((STCAP,), jnp.int32),
        ],
    )(_p1_kernel)
    p2 = functools.partial(
        pl.kernel,
        mesh=mesh,
        out_type=jax.ShapeDtypeStruct((BATCH,), jnp.float32),
        compiler_params=pltpu.CompilerParams(
            needs_layout_passes=False, use_tc_tiling_on_sc=False),
        scratch_types=[
            pltpu.VMEM((ROWS_PER_W,), jnp.int32),
            pltpu.VMEM((NBUF, 2, CHUNK, 2 * DIM), jnp.float32),
            pltpu.VMEM((NBUF, 2, CHUNK, DIM), jnp.float32),
            pltpu.VMEM((ROWS_PER_W,), jnp.float32),
            pltpu.SemaphoreType.DMA((NBUF,)),
        ],
    )(_p2_kernel)

    ere3 = entity_re.T.reshape(8, 8, NENT)
    eim3 = entity_im.T.reshape(8, 8, NENT)
    hst, tst = p1(h, t, ere3, eim3)
    return p2(r, hst, tst, rel_re, rel_im)


def kernel(h, r, t, entity_re, entity_im, rel_re, rel_im):
    return _score(h, r, t, entity_re, entity_im, rel_re, rel_im)


# speedup vs baseline: 4.1515x; 1.5129x over previous
import functools

import jax
import jax.numpy as jnp
from jax import lax
from jax.experimental import pallas as pl
from jax.experimental.pallas import tpu as pltpu
from jax.experimental.pallas import tpu_sc as plsc

DIM = 64
BATCH = 16384
NENT = 1000000
LANES = 16
NUM_CORES = 2
NUM_SUBCORES = 16
NUM_WORKERS = NUM_CORES * NUM_SUBCORES
ROWS_PER_W = BATCH // NUM_WORKERS
CHUNK = 128
NUM_CHUNKS = ROWS_PER_W // CHUNK
NUM_GROUPS = DIM // LANES
NBUF = 2

BW = 256
NBLKS = (NENT + BW - 1) // BW
LAST_BW = NENT - (NBLKS - 1) * BW
BPT = (NBLKS + NUM_WORKERS - 1) // NUM_WORKERS
STCAP = 64
DUMMY = BATCH


SLICE = 2048


def _p1_kernel(h_hbm, t_hbm, ere3, eim3,
               hst_hbm, tst_hbm,
               slice_v, hlist_v, tlist_v,
               eb_v, ib_v, tmp_v,
               hstg_v, tstg_v, hbl_v, tbl_v, sems):
    wid = lax.axis_index("s") * NUM_CORES + lax.axis_index("c")
    b0 = wid * BPT
    nblk = jnp.minimum(BPT, NBLKS - b0)
    lane_iota = lax.iota(jnp.int32, LANES)

    def build(arr_hbm, list_v):
        cnt = 0
        for s in range(BATCH // SLICE):
            pltpu.sync_copy(arr_hbm.at[pl.ds(s * SLICE, SLICE)], slice_v)

            def sb(i, cnt, s=s):
                e = slice_v[pl.ds(i * LANES, LANES)]
                blk = e >> 8
                m = (blk >= b0) & (blk < b0 + nblk)
                cnt_i = jnp.sum(m.astype(jnp.int32))
                enc = (((e - b0 * BW) << 14)
                       | (s * SLICE + i * LANES + lane_iota))
                plsc.store_compressed(list_v.at[pl.ds(cnt, LANES)], enc,
                                      mask=m)
                return cnt + cnt_i

            cnt = lax.fori_loop(0, SLICE // LANES, sb, cnt)
        return cnt

    nh = build(h_hbm, hlist_v)
    nt = build(t_hbm, tlist_v)

    dummy_vec = jnp.full((LANES,), DUMMY, jnp.int32)
    for q in range(STCAP // LANES):
        hbl_v[pl.ds(q * LANES, LANES)] = dummy_vec
        tbl_v[pl.ds(q * LANES, LANES)] = dummy_vec

    def serve(list_v, n, j, buf, col_base, stg_v, bl_v, out_hbm, cnt0):
        niter = (n + LANES - 1) >> 4

        def vs(i, cnt):
            encs = list_v[pl.ds(i * LANES, LANES)]
            valid = (i * LANES + lane_iota) < n
            mm = valid & ((encs >> 22) == j)
            cm = jnp.sum(mm.astype(jnp.int32))
            plsc.store_compressed(tmp_v.at[pl.ds(0, LANES)], encs, mask=mm)

            def hb(k, cnt):
                es = tmp_v[pl.ds(k, LANES)]
                enc_s = jnp.sum(jnp.where(lane_iota == 0, es, 0))
                b = enc_s & (BATCH - 1)
                col = ((enc_s >> 14) & (BW - 1)) + col_base
                colv = jnp.full((LANES,), 0, jnp.int32) + col
                slot = cnt & (STCAP - 1)
                for g in range(NUM_GROUPS):
                    d = g * LANES + lane_iota
                    idx = [d >> 3, d & 7, colv]
                    stg_v[slot, pl.ds(g * LANES, LANES)] = \
                        plsc.load_gather(eb_v.at[buf], idx)
                    stg_v[slot, pl.ds(DIM + g * LANES, LANES)] = \
                        plsc.load_gather(ib_v.at[buf], idx)
                q = slot >> 4
                rmod = slot & (LANES - 1)
                w = bl_v[pl.ds(q * LANES, LANES)]
                bl_v[pl.ds(q * LANES, LANES)] = \
                    jnp.where(lane_iota == rmod, b, w)
                cnt = cnt + 1

                @pl.when((cnt & (STCAP - 1)) == 0)
                def _flush():
                    pltpu.sync_copy(stg_v, out_hbm.at[bl_v])
                    for q2 in range(STCAP // LANES):
                        bl_v[pl.ds(q2 * LANES, LANES)] = dummy_vec

                return cnt

            return lax.fori_loop(0, cm, hb, cnt)

        return lax.fori_loop(0, niter, vs, cnt0)

    pad_edge = (NENT + 127) // 128 * 128

    def fire_blk(j, buf):
        jg = b0 + j
        start = jnp.where(jg == NBLKS - 1, pad_edge - BW, jg * BW)
        pltpu.async_copy(ere3.at[:, :, pl.ds(start, BW)], eb_v.at[buf],
                         sems.at[buf])
        pltpu.async_copy(eim3.at[:, :, pl.ds(start, BW)], ib_v.at[buf],
                         sems.at[buf])

    def drain_blk(buf):
        for _ in range(2):
            pltpu.make_async_copy(ere3.at[:, :, pl.ds(0, BW)], eb_v.at[buf],
                                  sems.at[buf]).wait()

    fire_blk(0, 0)

    def bb(j, carry):
        ch, ct = carry
        jg = b0 + j
        buf = j & 1
        drain_blk(buf)

        @pl.when(j + 1 < nblk)
        def _prefetch():
            fire_blk(j + 1, 1 - buf)

        col_base = jnp.where(jg == NBLKS - 1,
                             BW - (pad_edge - (NBLKS - 1) * BW), 0)
        ch = serve(hlist_v, nh, j, buf, col_base, hstg_v, hbl_v, hst_hbm, ch)
        ct = serve(tlist_v, nt, j, buf, col_base, tstg_v, tbl_v, tst_hbm, ct)
        return (ch, ct)

    ch, ct = lax.fori_loop(0, nblk, bb, (0, 0))

    @pl.when((ch & (STCAP - 1)) != 0)
    def _tail_h():
        pltpu.sync_copy(hstg_v, hst_hbm.at[hbl_v])

    @pl.when((ct & (STCAP - 1)) != 0)
    def _tail_t():
        pltpu.sync_copy(tstg_v, tst_hbm.at[tbl_v])


def _p2_fire(c, buf, base, hst_hbm, tst_hbm, rre_hbm, rim_hbm,
             ridx_v, ent_v, rel_v, sems):
    off = c * CHUNK
    rsl = ridx_v.at[pl.ds(off, CHUNK)]
    sl = pl.ds(base + off, CHUNK)
    pltpu.async_copy(hst_hbm.at[sl], ent_v.at[buf, 0], sems.at[buf])
    pltpu.async_copy(tst_hbm.at[sl], ent_v.at[buf, 1], sems.at[buf])
    pltpu.async_copy(rre_hbm.at[rsl], rel_v.at[buf, 0], sems.at[buf])
    pltpu.async_copy(rim_hbm.at[rsl], rel_v.at[buf, 1], sems.at[buf])


def _p2_drain(buf, hst_hbm, rre_hbm, ent_v, rel_v, sems):
    for k in range(2):
        pltpu.make_async_copy(hst_hbm.at[pl.ds(0, CHUNK)], ent_v.at[buf, k],
                              sems.at[buf]).wait()
    for k in range(2):
        pltpu.make_async_copy(rre_hbm.at[pl.ds(0, CHUNK)], rel_v.at[buf, k],
                              sems.at[buf]).wait()


def _p2_kernel(r_hbm, hst_hbm, tst_hbm, rre_hbm, rim_hbm,
               out_hbm, ridx_v, ent_v, rel_v, out_v, sems):
    wid = lax.axis_index("s") * NUM_CORES + lax.axis_index("c")
    base = wid * ROWS_PER_W
    lane_iota = lax.iota(jnp.int32, LANES)

    pltpu.sync_copy(r_hbm.at[pl.ds(base, ROWS_PER_W)], ridx_v)

    args = (base, hst_hbm, tst_hbm, rre_hbm, rim_hbm,
            ridx_v, ent_v, rel_v, sems)
    _p2_fire(0, 0, *args)

    for c in range(NUM_CHUNKS):
        buf = c % NBUF
        _p2_drain(buf, hst_hbm, rre_hbm, ent_v, rel_v, sems)
        if c + 1 < NUM_CHUNKS:
            _p2_fire(c + 1, (c + 1) % NBUF, *args)

        hv = ent_v.at[buf, 0]
        tv = ent_v.at[buf, 1]
        rr_v = rel_v.at[buf, 0]
        ri_v = rel_v.at[buf, 1]
        off = c * CHUNK

        def group_body(g16, _, hv=hv, tv=tv, rr_v=rr_v, ri_v=ri_v, off=off):
            rows = g16 * LANES + lane_iota

            def dim_body(d, acc):
                cols = jnp.full((LANES,), 0, jnp.int32) + d
                hr = plsc.load_gather(hv, [rows, cols])
                hi = plsc.load_gather(hv, [rows, cols + DIM])
                tr = plsc.load_gather(tv, [rows, cols])
                ti = plsc.load_gather(tv, [rows, cols + DIM])
                rr = plsc.load_gather(rr_v, [rows, cols])
                ri = plsc.load_gather(ri_v, [rows, cols])
                return (acc + rr * (hr * tr + hi * ti)
                        + ri * (hr * ti - hi * tr))

            scores = lax.fori_loop(0, DIM, dim_body,
                                   jnp.zeros((LANES,), jnp.float32))
            out_v[pl.ds(off + g16 * LANES, LANES)] = scores
            return 0

        lax.fori_loop(0, CHUNK // LANES, group_body, 0)

    pltpu.sync_copy(out_v, out_hbm.at[pl.ds(base, ROWS_PER_W)])


@functools.partial(jax.jit)
def _score(h, r, t, entity_re, entity_im, rel_re, rel_im):
    mesh = plsc.VectorSubcoreMesh(core_axis_name="c", subcore_axis_name="s")
    stage = jax.ShapeDtypeStruct((BATCH + STCAP, 2 * DIM), jnp.float32)
    p1 = functools.partial(
        pl.kernel,
        mesh=mesh,
        out_type=[stage, stage],
        compiler_params=pltpu.CompilerParams(needs_layout_passes=False),
        scratch_types=[
            pltpu.VMEM((SLICE,), jnp.int32),
            pltpu.VMEM((BATCH + LANES,), jnp.int32),
            pltpu.VMEM((BATCH + LANES,), jnp.int32),
            pltpu.VMEM((2, 8, 8, BW), jnp.float32),
            pltpu.VMEM((2, 8, 8, BW), jnp.float32),
            pltpu.VMEM((2 * LANES - 1,), jnp.int32),
            pltpu.VMEM((STCAP, 2 * DIM), jnp.float32),
            pltpu.VMEM((STCAP, 2 * DIM), jnp.float32),
            pltpu.VMEM((STCAP,), jnp.int32),
            pltpu.VMEM((STCAP,), jnp.int32),
            pltpu.SemaphoreType.DMA((2,)),
        ],
    )(_p1_kernel)
    p2 = functools.partial(
        pl.kernel,
        mesh=mesh,
        out_type=jax.ShapeDtypeStruct((BATCH,), jnp.float32),
        compiler_params=pltpu.CompilerParams(
            needs_layout_passes=False, use_tc_tiling_on_sc=False),
        scratch_types=[
            pltpu.VMEM((ROWS_PER_W,), jnp.int32),
            pltpu.VMEM((NBUF, 2, CHUNK, 2 * DIM), jnp.float32),
            pltpu.VMEM((NBUF, 2, CHUNK, DIM), jnp.float32),
            pltpu.VMEM((ROWS_PER_W,), jnp.float32),
            pltpu.SemaphoreType.DMA((NBUF,)),
        ],
    )(_p2_kernel)

    ere3 = entity_re.T.reshape(8, 8, NENT)
    eim3 = entity_im.T.reshape(8, 8, NENT)
    hst, tst = p1(h, t, ere3, eim3)
    return p2(r, hst, tst, rel_re, rel_im)


def kernel(h, r, t, entity_re, entity_im, rel_re, rel_im):
    return _score(h, r, t, entity_re, entity_im, rel_re, rel_im)


# speedup vs baseline: 4.3391x; 1.0452x over previous
import functools

import jax
import jax.numpy as jnp
from jax import lax
from jax.experimental import pallas as pl
from jax.experimental.pallas import tpu as pltpu
from jax.experimental.pallas import tpu_sc as plsc

DIM = 64
BATCH = 16384
NENT = 1000000
LANES = 16
NUM_CORES = 2
NUM_SUBCORES = 16
NUM_WORKERS = NUM_CORES * NUM_SUBCORES
ROWS_PER_W = BATCH // NUM_WORKERS
CHUNK = 128
NUM_CHUNKS = ROWS_PER_W // CHUNK
NUM_GROUPS = DIM // LANES
NBUF = 2

BW = 128
NBLKS = (NENT + BW - 1) // BW
BPT = (NBLKS + NUM_WORKERS - 1) // NUM_WORKERS
NBKT = 16
STCAP = 64
DUMMY = BATCH


SLICE = 2048


def _p1_kernel(h_hbm, t_hbm, ere3, eim3,
               hst_hbm, tst_hbm,
               slice_v, hlist_v, tlist_v, hbkt_v, tbkt_v,
               eb_v, ib_v, tmp_v,
               hstg_v, tstg_v, hbl_v, tbl_v, sems):
    wid = lax.axis_index("s") * NUM_CORES + lax.axis_index("c")
    b0 = wid * BPT
    nblk = jnp.minimum(BPT, NBLKS - b0)
    lane_iota = lax.iota(jnp.int32, LANES)

    def build(arr_hbm, list_v):
        cnt = 0
        for s in range(BATCH // SLICE):
            pltpu.sync_copy(arr_hbm.at[pl.ds(s * SLICE, SLICE)], slice_v)

            def sb(i, cnt, s=s):
                e = slice_v[pl.ds(i * LANES, LANES)]
                blk = e >> 7
                m = (blk >= b0) & (blk < b0 + nblk)
                cnt_i = jnp.sum(m.astype(jnp.int32))
                enc = (((e - b0 * BW) << 14)
                       | (s * SLICE + i * LANES + lane_iota))
                plsc.store_compressed(list_v.at[pl.ds(cnt, LANES)], enc,
                                      mask=m)
                return cnt + cnt_i

            cnt = lax.fori_loop(0, SLICE // LANES, sb, cnt)
        return cnt

    nh = build(h_hbm, hlist_v)
    nt = build(t_hbm, tlist_v)

    def bucketize(list_v, n, bkt_v):
        offs = []
        cnt = 0
        niter = (n + LANES - 1) >> 4
        for u in range(NBKT):
            offs.append(cnt)

            def bs(i, cnt, u=u):
                encs = list_v[pl.ds(i * LANES, LANES)]
                valid = (i * LANES + lane_iota) < n
                m = valid & ((encs >> 25) == u)
                ci = jnp.sum(m.astype(jnp.int32))
                plsc.store_compressed(bkt_v.at[pl.ds(cnt, LANES)], encs,
                                      mask=m)
                return cnt + ci

            cnt = lax.fori_loop(0, niter, bs, cnt)
        offs.append(cnt)
        return offs

    hoffs = bucketize(hlist_v, nh, hbkt_v)
    toffs = bucketize(tlist_v, nt, tbkt_v)

    dummy_vec = jnp.full((LANES,), DUMMY, jnp.int32)
    for q in range(STCAP // LANES):
        hbl_v[pl.ds(q * LANES, LANES)] = dummy_vec
        tbl_v[pl.ds(q * LANES, LANES)] = dummy_vec

    def serve(bkt_v, start, end, j, buf, stg_v, bl_v, out_hbm, cnt0):
        i0 = start >> 4
        i1 = (end + LANES - 1) >> 4

        def vs(i, cnt):
            encs = bkt_v[pl.ds(i * LANES, LANES)]
            pos = i * LANES + lane_iota
            valid = (pos >= start) & (pos < end)
            mm = valid & ((encs >> 21) == j)
            cm = jnp.sum(mm.astype(jnp.int32))
            plsc.store_compressed(tmp_v.at[pl.ds(0, LANES)], encs, mask=mm)

            def hb(k, cnt):
                es = tmp_v[pl.ds(k, LANES)]
                enc_s = jnp.sum(jnp.where(lane_iota == 0, es, 0))
                b = enc_s & (BATCH - 1)
                col = (enc_s >> 14) & (BW - 1)
                colv = jnp.full((LANES,), 0, jnp.int32) + col
                slot = cnt & (STCAP - 1)
                for g in range(NUM_GROUPS):
                    d = g * LANES + lane_iota
                    idx = [d >> 3, d & 7, colv]
                    stg_v[slot, pl.ds(g * LANES, LANES)] = \
                        plsc.load_gather(eb_v.at[buf], idx)
                    stg_v[slot, pl.ds(DIM + g * LANES, LANES)] = \
                        plsc.load_gather(ib_v.at[buf], idx)
                q = slot >> 4
                rmod = slot & (LANES - 1)
                w = bl_v[pl.ds(q * LANES, LANES)]
                bl_v[pl.ds(q * LANES, LANES)] = \
                    jnp.where(lane_iota == rmod, b, w)
                cnt = cnt + 1

                @pl.when((cnt & (STCAP - 1)) == 0)
                def _flush():
                    pltpu.sync_copy(stg_v, out_hbm.at[bl_v])
                    for q2 in range(STCAP // LANES):
                        bl_v[pl.ds(q2 * LANES, LANES)] = dummy_vec

                return cnt

            return lax.fori_loop(0, cm, hb, cnt)

        return lax.fori_loop(i0, i1, vs, cnt0)

    def fire_blk(j, buf):
        pltpu.async_copy(ere3.at[:, :, pl.ds((b0 + j) * BW, BW)],
                         eb_v.at[buf], sems.at[buf])
        pltpu.async_copy(eim3.at[:, :, pl.ds((b0 + j) * BW, BW)],
                         ib_v.at[buf], sems.at[buf])

    def drain_blk(buf):
        for _ in range(2):
            pltpu.make_async_copy(ere3.at[:, :, pl.ds(0, BW)], eb_v.at[buf],
                                  sems.at[buf]).wait()

    def pick(offs, u):
        lo, hi = offs[0], offs[NBKT]
        lo = functools.reduce(
            lambda a, k: jnp.where(u == k, offs[k], a), range(1, NBKT), lo)
        hi = functools.reduce(
            lambda a, k: jnp.where(u == k, offs[k + 1], a), range(NBKT - 1),
            hi)
        return lo, hi

    fire_blk(0, 0)

    def bb(j, carry):
        ch, ct = carry
        buf = j & 1
        drain_blk(buf)

        @pl.when(j + 1 < nblk)
        def _prefetch():
            fire_blk(j + 1, 1 - buf)

        u = j >> 4
        hs, he = pick(hoffs, u)
        ts, te = pick(toffs, u)
        ch = serve(hbkt_v, hs, he, j, buf, hstg_v, hbl_v, hst_hbm, ch)
        ct = serve(tbkt_v, ts, te, j, buf, tstg_v, tbl_v, tst_hbm, ct)
        return (ch, ct)

    ch, ct = lax.fori_loop(0, nblk, bb, (0, 0))

    @pl.when((ch & (STCAP - 1)) != 0)
    def _tail_h():
        pltpu.sync_copy(hstg_v, hst_hbm.at[hbl_v])

    @pl.when((ct & (STCAP - 1)) != 0)
    def _tail_t():
        pltpu.sync_copy(tstg_v, tst_hbm.at[tbl_v])


def _p2_fire(c, buf, base, hst_hbm, tst_hbm, rre_hbm, rim_hbm,
             ridx_v, ent_v, rel_v, sems):
    off = c * CHUNK
    rsl = ridx_v.at[pl.ds(off, CHUNK)]
    sl = pl.ds(base + off, CHUNK)
    pltpu.async_copy(hst_hbm.at[sl], ent_v.at[buf, 0], sems.at[buf])
    pltpu.async_copy(tst_hbm.at[sl], ent_v.at[buf, 1], sems.at[buf])
    pltpu.async_copy(rre_hbm.at[rsl], rel_v.at[buf, 0], sems.at[buf])
    pltpu.async_copy(rim_hbm.at[rsl], rel_v.at[buf, 1], sems.at[buf])


def _p2_drain(buf, hst_hbm, rre_hbm, ent_v, rel_v, sems):
    for k in range(2):
        pltpu.make_async_copy(hst_hbm.at[pl.ds(0, CHUNK)], ent_v.at[buf, k],
                              sems.at[buf]).wait()
    for k in range(2):
        pltpu.make_async_copy(rre_hbm.at[pl.ds(0, CHUNK)], rel_v.at[buf, k],
                              sems.at[buf]).wait()


def _p2_kernel(r_hbm, hst_hbm, tst_hbm, rre_hbm, rim_hbm,
               out_hbm, ridx_v, ent_v, rel_v, out_v, sems):
    wid = lax.axis_index("s") * NUM_CORES + lax.axis_index("c")
    base = wid * ROWS_PER_W
    lane_iota = lax.iota(jnp.int32, LANES)

    pltpu.sync_copy(r_hbm.at[pl.ds(base, ROWS_PER_W)], ridx_v)

    args = (base, hst_hbm, tst_hbm, rre_hbm, rim_hbm,
            ridx_v, ent_v, rel_v, sems)
    _p2_fire(0, 0, *args)

    for c in range(NUM_CHUNKS):
        buf = c % NBUF
        _p2_drain(buf, hst_hbm, rre_hbm, ent_v, rel_v, sems)
        if c + 1 < NUM_CHUNKS:
            _p2_fire(c + 1, (c + 1) % NBUF, *args)

        hv = ent_v.at[buf, 0]
        tv = ent_v.at[buf, 1]
        rr_v = rel_v.at[buf, 0]
        ri_v = rel_v.at[buf, 1]
        off = c * CHUNK

        def group_body(g16, _, hv=hv, tv=tv, rr_v=rr_v, ri_v=ri_v, off=off):
            rows = g16 * LANES + lane_iota

            def dim_body(d, acc):
                cols = jnp.full((LANES,), 0, jnp.int32) + d
                hr = plsc.load_gather(hv, [rows, cols])
                hi = plsc.load_gather(hv, [rows, cols + DIM])
                tr = plsc.load_gather(tv, [rows, cols])
                ti = plsc.load_gather(tv, [rows, cols + DIM])
                rr = plsc.load_gather(rr_v, [rows, cols])
                ri = plsc.load_gather(ri_v, [rows, cols])
                return (acc + rr * (hr * tr + hi * ti)
                        + ri * (hr * ti - hi * tr))

            scores = lax.fori_loop(0, DIM, dim_body,
                                   jnp.zeros((LANES,), jnp.float32))
            out_v[pl.ds(off + g16 * LANES, LANES)] = scores
            return 0

        lax.fori_loop(0, CHUNK // LANES, group_body, 0)

    pltpu.sync_copy(out_v, out_hbm.at[pl.ds(base, ROWS_PER_W)])


@functools.partial(jax.jit)
def _score(h, r, t, entity_re, entity_im, rel_re, rel_im):
    mesh = plsc.VectorSubcoreMesh(core_axis_name="c", subcore_axis_name="s")
    stage = jax.ShapeDtypeStruct((BATCH + STCAP, 2 * DIM), jnp.float32)
    p1 = functools.partial(
        pl.kernel,
        mesh=mesh,
        out_type=[stage, stage],
        compiler_params=pltpu.CompilerParams(needs_layout_passes=False),
        scratch_types=[
            pltpu.VMEM((SLICE,), jnp.int32),
            pltpu.VMEM((BATCH + LANES,), jnp.int32),
            pltpu.VMEM((BATCH + LANES,), jnp.int32),
            pltpu.VMEM((BATCH + LANES,), jnp.int32),
            pltpu.VMEM((BATCH + LANES,), jnp.int32),
            pltpu.VMEM((2, 8, 8, BW), jnp.float32),
            pltpu.VMEM((2, 8, 8, BW), jnp.float32),
            pltpu.VMEM((2 * LANES - 1,), jnp.int32),
            pltpu.VMEM((STCAP, 2 * DIM), jnp.float32),
            pltpu.VMEM((STCAP, 2 * DIM), jnp.float32),
            pltpu.VMEM((STCAP,), jnp.int32),
            pltpu.VMEM((STCAP,), jnp.int32),
            pltpu.SemaphoreType.DMA((2,)),
        ],
    )(_p1_kernel)
    p2 = functools.partial(
        pl.kernel,
        mesh=mesh,
        out_type=jax.ShapeDtypeStruct((BATCH,), jnp.float32),
        compiler_params=pltpu.CompilerParams(
            needs_layout_passes=False, use_tc_tiling_on_sc=False),
        scratch_types=[
            pltpu.VMEM((ROWS_PER_W,), jnp.int32),
            pltpu.VMEM((NBUF, 2, CHUNK, 2 * DIM), jnp.float32),
            pltpu.VMEM((NBUF, 2, CHUNK, DIM), jnp.float32),
            pltpu.VMEM((ROWS_PER_W,), jnp.float32),
            pltpu.SemaphoreType.DMA((NBUF,)),
        ],
    )(_p2_kernel)

    ere3 = entity_re.T.reshape(8, 8, NENT)
    eim3 = entity_im.T.reshape(8, 8, NENT)
    hst, tst = p1(h, t, ere3, eim3)
    return p2(r, hst, tst, rel_re, rel_im)


def kernel(h, r, t, entity_re, entity_im, rel_re, rel_im):
    return _score(h, r, t, entity_re, entity_im, rel_re, rel_im)


# speedup vs baseline: 5.4072x; 1.2462x over previous
import functools

import jax
import jax.numpy as jnp
from jax import lax
from jax.experimental import pallas as pl
from jax.experimental.pallas import tpu as pltpu
from jax.experimental.pallas import tpu_sc as plsc

DIM = 64
BATCH = 16384
NENT = 1000000
LANES = 16
NUM_CORES = 2
NUM_SUBCORES = 16
NUM_WORKERS = NUM_CORES * NUM_SUBCORES
ROWS_PER_W = BATCH // NUM_WORKERS
CHUNK = 128
NUM_CHUNKS = ROWS_PER_W // CHUNK
NUM_GROUPS = DIM // LANES
NBUF = 2

BW = 256
NBLKS = (NENT + BW - 1) // BW
BPT = (NBLKS + NUM_WORKERS - 1) // NUM_WORKERS
NBKT = 8
PAD_EDGE = (NENT + 127) // 128 * 128
STCAP = 32
DUMMY = BATCH


SLICE = 2048


def _p1_kernel(h_hbm, t_hbm, ere3, eim3,
               hst_hbm, tst_hbm,
               slice_v, hlist_v, tlist_v, hbkt_v,
               eb_v, ib_v, tmp_v,
               hstg_v, tstg_v, hbl_v, tbl_v, sems):
    wid = lax.axis_index("s") * NUM_CORES + lax.axis_index("c")
    b0 = wid * BPT
    nblk = jnp.minimum(BPT, NBLKS - b0)
    lane_iota = lax.iota(jnp.int32, LANES)

    def build(arr_hbm, list_v):
        cnt = 0
        for s in range(BATCH // SLICE):
            pltpu.sync_copy(arr_hbm.at[pl.ds(s * SLICE, SLICE)], slice_v)

            def sb(i, cnt, s=s):
                e = slice_v[pl.ds(i * LANES, LANES)]
                blk = e >> 8
                m = (blk >= b0) & (blk < b0 + nblk)
                cnt_i = jnp.sum(m.astype(jnp.int32))
                enc = (((e - b0 * BW) << 14)
                       | (s * SLICE + i * LANES + lane_iota))
                plsc.store_compressed(list_v.at[pl.ds(cnt, LANES)], enc,
                                      mask=m)
                return cnt + cnt_i

            cnt = lax.fori_loop(0, SLICE // LANES, sb, cnt)
        return cnt

    nh = build(h_hbm, hlist_v)
    nt = build(t_hbm, tlist_v)

    def bucketize(list_v, n, bkt_v):
        offs = []
        cnt = 0
        niter = (n + LANES - 1) >> 4
        for u in range(NBKT):
            offs.append(cnt)

            def bs(i, cnt, u=u):
                encs = list_v[pl.ds(i * LANES, LANES)]
                valid = (i * LANES + lane_iota) < n
                m = valid & ((encs >> 26) == u)
                ci = jnp.sum(m.astype(jnp.int32))
                plsc.store_compressed(bkt_v.at[pl.ds(cnt, LANES)], encs,
                                      mask=m)
                return cnt + ci

            cnt = lax.fori_loop(0, niter, bs, cnt)
        offs.append(cnt)
        return offs

    hoffs = bucketize(hlist_v, nh, hbkt_v)
    toffs = bucketize(tlist_v, nt, hlist_v)
    tbkt_v = hlist_v

    dummy_vec = jnp.full((LANES,), DUMMY, jnp.int32)
    for q in range(STCAP // LANES):
        hbl_v[pl.ds(q * LANES, LANES)] = dummy_vec
        tbl_v[pl.ds(q * LANES, LANES)] = dummy_vec

    def serve(bkt_v, start, end, j, buf, col_base, stg_v, bl_v, out_hbm,
              cnt0):
        i0 = start >> 4
        i1 = (end + LANES - 1) >> 4

        def vs(i, cnt):
            encs = bkt_v[pl.ds(i * LANES, LANES)]
            pos = i * LANES + lane_iota
            valid = (pos >= start) & (pos < end)
            mm = valid & ((encs >> 22) == j)
            cm = jnp.sum(mm.astype(jnp.int32))
            plsc.store_compressed(tmp_v.at[pl.ds(0, LANES)], encs, mask=mm)

            def hb(k, cnt):
                es = tmp_v[pl.ds(k, LANES)]
                enc_s = jnp.sum(jnp.where(lane_iota == 0, es, 0))
                b = enc_s & (BATCH - 1)
                col = ((enc_s >> 14) & (BW - 1)) + col_base
                colv = jnp.full((LANES,), 0, jnp.int32) + col
                slot = cnt & (STCAP - 1)
                for g in range(NUM_GROUPS):
                    d = g * LANES + lane_iota
                    idx = [d >> 3, d & 7, colv]
                    stg_v[slot, pl.ds(g * LANES, LANES)] = \
                        plsc.load_gather(eb_v.at[buf], idx)
                    stg_v[slot, pl.ds(DIM + g * LANES, LANES)] = \
                        plsc.load_gather(ib_v.at[buf], idx)
                q = slot >> 4
                rmod = slot & (LANES - 1)
                w = bl_v[pl.ds(q * LANES, LANES)]
                bl_v[pl.ds(q * LANES, LANES)] = \
                    jnp.where(lane_iota == rmod, b, w)
                cnt = cnt + 1

                @pl.when((cnt & (STCAP - 1)) == 0)
                def _flush():
                    pltpu.sync_copy(stg_v, out_hbm.at[bl_v])
                    for q2 in range(STCAP // LANES):
                        bl_v[pl.ds(q2 * LANES, LANES)] = dummy_vec

                return cnt

            return lax.fori_loop(0, cm, hb, cnt)

        return lax.fori_loop(i0, i1, vs, cnt0)

    def fire_blk(j, buf):
        jg = b0 + j
        start = jnp.where(jg == NBLKS - 1, PAD_EDGE - BW, jg * BW)
        pltpu.async_copy(ere3.at[:, :, pl.ds(start, BW)],
                         eb_v.at[buf], sems.at[buf])
        pltpu.async_copy(eim3.at[:, :, pl.ds(start, BW)],
                         ib_v.at[buf], sems.at[buf])

    def drain_blk(buf):
        for _ in range(2):
            pltpu.make_async_copy(ere3.at[:, :, pl.ds(0, BW)], eb_v.at[buf],
                                  sems.at[buf]).wait()

    def pick(offs, u):
        lo, hi = offs[0], offs[NBKT]
        lo = functools.reduce(
            lambda a, k: jnp.where(u == k, offs[k], a), range(1, NBKT), lo)
        hi = functools.reduce(
            lambda a, k: jnp.where(u == k, offs[k + 1], a), range(NBKT - 1),
            hi)
        return lo, hi

    fire_blk(0, 0)

    def bb(j, carry):
        ch, ct = carry
        buf = j & 1
        drain_blk(buf)

        @pl.when(j + 1 < nblk)
        def _prefetch():
            fire_blk(j + 1, 1 - buf)

        u = j >> 4
        hs, he = pick(hoffs, u)
        ts, te = pick(toffs, u)
        col_base = jnp.where(b0 + j == NBLKS - 1,
                             BW - (PAD_EDGE - (NBLKS - 1) * BW), 0)
        ch = serve(hbkt_v, hs, he, j, buf, col_base, hstg_v, hbl_v,
                   hst_hbm, ch)
        ct = serve(tbkt_v, ts, te, j, buf, col_base, tstg_v, tbl_v,
                   tst_hbm, ct)
        return (ch, ct)

    ch, ct = lax.fori_loop(0, nblk, bb, (0, 0))

    @pl.when((ch & (STCAP - 1)) != 0)
    def _tail_h():
        pltpu.sync_copy(hstg_v, hst_hbm.at[hbl_v])

    @pl.when((ct & (STCAP - 1)) != 0)
    def _tail_t():
        pltpu.sync_copy(tstg_v, tst_hbm.at[tbl_v])


def _p2_fire(c, buf, base, hst_hbm, tst_hbm, rre_hbm, rim_hbm,
             ridx_v, ent_v, rel_v, sems):
    off = c * CHUNK
    rsl = ridx_v.at[pl.ds(off, CHUNK)]
    sl = pl.ds(base + off, CHUNK)
    pltpu.async_copy(hst_hbm.at[sl], ent_v.at[buf, 0], sems.at[buf])
    pltpu.async_copy(tst_hbm.at[sl], ent_v.at[buf, 1], sems.at[buf])
    pltpu.async_copy(rre_hbm.at[rsl], rel_v.at[buf, 0], sems.at[buf])
    pltpu.async_copy(rim_hbm.at[rsl], rel_v.at[buf, 1], sems.at[buf])


def _p2_drain(buf, hst_hbm, rre_hbm, ent_v, rel_v, sems):
    for k in range(2):
        pltpu.make_async_copy(hst_hbm.at[pl.ds(0, CHUNK)], ent_v.at[buf, k],
                              sems.at[buf]).wait()
    for k in range(2):
        pltpu.make_async_copy(rre_hbm.at[pl.ds(0, CHUNK)], rel_v.at[buf, k],
                              sems.at[buf]).wait()


def _p2_kernel(r_hbm, hst_hbm, tst_hbm, rre_hbm, rim_hbm,
               out_hbm, ridx_v, ent_v, rel_v, out_v, sems):
    wid = lax.axis_index("s") * NUM_CORES + lax.axis_index("c")
    base = wid * ROWS_PER_W
    lane_iota = lax.iota(jnp.int32, LANES)

    pltpu.sync_copy(r_hbm.at[pl.ds(base, ROWS_PER_W)], ridx_v)

    args = (base, hst_hbm, tst_hbm, rre_hbm, rim_hbm,
            ridx_v, ent_v, rel_v, sems)
    _p2_fire(0, 0, *args)

    for c in range(NUM_CHUNKS):
        buf = c % NBUF
        _p2_drain(buf, hst_hbm, rre_hbm, ent_v, rel_v, sems)
        if c + 1 < NUM_CHUNKS:
            _p2_fire(c + 1, (c + 1) % NBUF, *args)

        hv = ent_v.at[buf, 0]
        tv = ent_v.at[buf, 1]
        rr_v = rel_v.at[buf, 0]
        ri_v = rel_v.at[buf, 1]
        off = c * CHUNK

        def group_body(g16, _, hv=hv, tv=tv, rr_v=rr_v, ri_v=ri_v, off=off):
            rows = g16 * LANES + lane_iota

            def dim_body(d, acc):
                cols = jnp.full((LANES,), 0, jnp.int32) + d
                hr = plsc.load_gather(hv, [rows, cols])
                hi = plsc.load_gather(hv, [rows, cols + DIM])
                tr = plsc.load_gather(tv, [rows, cols])
                ti = plsc.load_gather(tv, [rows, cols + DIM])
                rr = plsc.load_gather(rr_v, [rows, cols])
                ri = plsc.load_gather(ri_v, [rows, cols])
                return (acc + rr * (hr * tr + hi * ti)
                        + ri * (hr * ti - hi * tr))

            scores = lax.fori_loop(0, DIM, dim_body,
                                   jnp.zeros((LANES,), jnp.float32))
            out_v[pl.ds(off + g16 * LANES, LANES)] = scores
            return 0

        lax.fori_loop(0, CHUNK // LANES, group_body, 0)

    pltpu.sync_copy(out_v, out_hbm.at[pl.ds(base, ROWS_PER_W)])


@functools.partial(jax.jit)
def _score(h, r, t, entity_re, entity_im, rel_re, rel_im):
    mesh = plsc.VectorSubcoreMesh(core_axis_name="c", subcore_axis_name="s")
    stage = jax.ShapeDtypeStruct((BATCH + STCAP, 2 * DIM), jnp.float32)
    p1 = functools.partial(
        pl.kernel,
        mesh=mesh,
        out_type=[stage, stage],
        compiler_params=pltpu.CompilerParams(needs_layout_passes=False),
        scratch_types=[
            pltpu.VMEM((SLICE,), jnp.int32),
            pltpu.VMEM((BATCH + LANES,), jnp.int32),
            pltpu.VMEM((BATCH + LANES,), jnp.int32),
            pltpu.VMEM((BATCH + LANES,), jnp.int32),
            pltpu.VMEM((2, 8, 8, BW), jnp.float32),
            pltpu.VMEM((2, 8, 8, BW), jnp.float32),
            pltpu.VMEM((2 * LANES - 1,), jnp.int32),
            pltpu.VMEM((STCAP, 2 * DIM), jnp.float32),
            pltpu.VMEM((STCAP, 2 * DIM), jnp.float32),
            pltpu.VMEM((STCAP,), jnp.int32),
            pltpu.VMEM((STCAP,), jnp.int32),
            pltpu.SemaphoreType.DMA((2,)),
        ],
    )(_p1_kernel)
    p2 = functools.partial(
        pl.kernel,
        mesh=mesh,
        out_type=jax.ShapeDtypeStruct((BATCH,), jnp.float32),
        compiler_params=pltpu.CompilerParams(
            needs_layout_passes=False, use_tc_tiling_on_sc=False),
        scratch_types=[
            pltpu.VMEM((ROWS_PER_W,), jnp.int32),
            pltpu.VMEM((NBUF, 2, CHUNK, 2 * DIM), jnp.float32),
            pltpu.VMEM((NBUF, 2, CHUNK, DIM), jnp.float32),
            pltpu.VMEM((ROWS_PER_W,), jnp.float32),
            pltpu.SemaphoreType.DMA((NBUF,)),
        ],
    )(_p2_kernel)

    ere3 = entity_re.T.reshape(8, 8, NENT)
    eim3 = entity_im.T.reshape(8, 8, NENT)
    hst, tst = p1(h, t, ere3, eim3)
    return p2(r, hst, tst, rel_re, rel_im)


def kernel(h, r, t, entity_re, entity_im, rel_re, rel_im):
    return _score(h, r, t, entity_re, entity_im, rel_re, rel_im)
